# baseline (speedup 1.0000x reference)
import functools

import jax
import jax.numpy as jnp
from jax import lax
from jax.experimental import pallas as pl
from jax.experimental.pallas import tpu as pltpu
from jax.experimental.pallas import tpu_sc as plsc

NC = 2
NS = 16
NW = NC * NS
L = 16
CH = 64
CP = 128


@functools.cache
def _build(B, H, W, C, N):
    HW = H * W
    full_per_b = N // CH
    tail = N - full_per_b * CH
    n_main = B * full_per_b
    assert n_main % NW == 0, (n_main, NW)
    per_w = n_main // NW
    assert tail % 8 == 0 and tail < CH
    sx = float(W) / float(W - 1)
    sy = float(H) / float(H - 1)

    mesh = plsc.VectorSubcoreMesh(
        core_axis_name="c", subcore_axis_name="s", num_cores=NC, num_subcores=NS
    )

    @functools.partial(
        pl.kernel,
        out_type=jax.ShapeDtypeStruct((B, N, C), jnp.float32),
        mesh=mesh,
        scratch_types=[
            pltpu.VMEM((2, CH), jnp.float32),
            pltpu.VMEM((2, CH), jnp.float32),
            pltpu.VMEM((2, 4, CH), jnp.int32),
            pltpu.VMEM((2, 4, CH), jnp.float32),
            pltpu.VMEM((2, 4 * CH, CP), jnp.float32),
            pltpu.VMEM((2, CH, C), jnp.float32),
            pltpu.SemaphoreType.DMA,
            pltpu.SemaphoreType.DMA,
            pltpu.SemaphoreType.DMA,
        ],
        compiler_params=pltpu.CompilerParams(use_tc_tiling_on_sc=False),
    )
    def interp(x_hbm, px_hbm, py_hbm, out_hbm, pxv, pyv, idxv, wv, rowsv, outv,
               psem, gsem, osem):
        wid = lax.axis_index("s") * NC + lax.axis_index("c")
        c0 = wid * per_w

        def chunk_coords(c):
            b = jnp.int32(0)
            for bb in range(1, B):
                b = b + jnp.where(c >= bb * full_per_b, 1, 0)
            lc = c - b * full_per_b
            return b, lc

        def fire_pos(t):
            c = c0 + t
            pbase = chunk_coords(c)[0] * (N - full_per_b * CH) + c * CH
            buf = t & 1
            pltpu.async_copy(px_hbm.at[pl.ds(pbase, CH)], pxv.at[buf], psem)
            pltpu.async_copy(py_hbm.at[pl.ds(pbase, CH)], pyv.at[buf], psem)

        def wait_pos():
            pltpu.make_async_copy(px_hbm.at[pl.ds(0, CH)], pxv.at[0], psem).wait()
            pltpu.make_async_copy(py_hbm.at[pl.ds(0, CH)], pyv.at[0], psem).wait()

        def weights_phase(t, size):
            buf = t & 1
            for g in range(size // L):
                sl = pl.ds(g * L, L)
                px = pxv[buf, sl]
                py = pyv[buf, sl]
                ix = px * sx - 0.5
                iy = py * sy - 0.5
                fx0 = (ix + 1.0).astype(jnp.int32).astype(jnp.float32) - 1.0
                fy0 = (iy + 1.0).astype(jnp.int32).astype(jnp.float32) - 1.0
                wx1 = ix - fx0
                wx0 = 1.0 - wx1
                wy1 = iy - fy0
                wy0 = 1.0 - wy1
                fx1 = fx0 + 1.0
                fy1 = fy0 + 1.0
                mx0 = (fx0 >= 0.0) & (fx0 <= W - 1.0)
                mx1 = (fx1 >= 0.0) & (fx1 <= W - 1.0)
                my0 = (fy0 >= 0.0) & (fy0 <= H - 1.0)
                my1 = (fy1 >= 0.0) & (fy1 <= H - 1.0)
                cx0 = jnp.clip(fx0.astype(jnp.int32), 0, W - 1)
                cx1 = jnp.clip(fx1.astype(jnp.int32), 0, W - 1)
                cy0 = jnp.clip(fy0.astype(jnp.int32), 0, H - 1)
                cy1 = jnp.clip(fy1.astype(jnp.int32), 0, H - 1)
                r0 = cy0 * W
                r1 = cy1 * W
                idxv[buf, 0, sl] = r0 + cx0
                idxv[buf, 1, sl] = r0 + cx1
                idxv[buf, 2, sl] = r1 + cx0
                idxv[buf, 3, sl] = r1 + cx1
                zero = jnp.zeros((L,), jnp.float32)
                wv[buf, 0, sl] = jnp.where(mx0 & my0, wx0 * wy0, zero)
                wv[buf, 1, sl] = jnp.where(mx1 & my0, wx1 * wy0, zero)
                wv[buf, 2, sl] = jnp.where(mx0 & my1, wx0 * wy1, zero)
                wv[buf, 3, sl] = jnp.where(mx1 & my1, wx1 * wy1, zero)

        def fire_gathers(t, size):
            buf = t & 1
            b = chunk_coords(c0 + t)[0]
            xb = x_hbm.at[b]
            for k in range(4):
                pltpu.async_copy(
                    xb.at[idxv.at[buf, k, pl.ds(0, size)]],
                    rowsv.at[buf, pl.ds(k * CH, size)],
                    gsem,
                )

        def wait_gathers(size):
            for _ in range(4):
                pltpu.make_async_copy(
                    x_hbm.at[0, pl.ds(0, size)],
                    rowsv.at[0, pl.ds(0, size)],
                    gsem,
                ).wait()

        def compute_phase(t, size):
            buf = t & 1
            b, lc = chunk_coords(c0 + t)

            def group_body(g, carry2):
                gsl = pl.ds(g * L, L)
                w00 = wv[buf, 0, gsl]
                w01 = wv[buf, 1, gsl]
                w10 = wv[buf, 2, gsl]
                w11 = wv[buf, 3, gsl]
                for j in range(L):
                    lanes = jnp.full((L,), j, jnp.int32)
                    b00 = w00.at[lanes].get(mode="promise_in_bounds")
                    b01 = w01.at[lanes].get(mode="promise_in_bounds")
                    b10 = w10.at[lanes].get(mode="promise_in_bounds")
                    b11 = w11.at[lanes].get(mode="promise_in_bounds")
                    p = g * L + j
                    for cc in range(C // L):
                        csl = pl.ds(cc * L, L)
                        acc = rowsv[buf, p, csl] * b00
                        acc += rowsv[buf, CH + p, csl] * b01
                        acc += rowsv[buf, 2 * CH + p, csl] * b10
                        acc += rowsv[buf, 3 * CH + p, csl] * b11
                        outv[buf, p, csl] = acc
                return carry2

            lax.fori_loop(0, size // L, group_body, 0)
            pltpu.async_copy(
                outv.at[buf, pl.ds(0, size)],
                out_hbm.at[b, pl.ds(lc * CH, size)],
                osem,
            )

        def wait_out():
            pltpu.make_async_copy(
                out_hbm.at[0, pl.ds(0, CH)], outv.at[0], osem
            ).wait()

        fire_pos(0)

        def pipe_body(t, carry):
            @pl.when(t < per_w)
            def _():
                wait_pos()
                weights_phase(t, CH)

                @pl.when(t + 1 < per_w)
                def _():
                    fire_pos(t + 1)

            @pl.when(t >= 1)
            def _():
                wait_gathers(CH)

            @pl.when(t < per_w)
            def _():
                fire_gathers(t, CH)

            @pl.when(t >= 2)
            def _():
                wait_out()

            @pl.when(t >= 1)
            def _():
                compute_phase(t - 1, CH)

            return carry

        lax.fori_loop(0, per_w + 1, pipe_body, 0)
        wait_out()

        if tail:
            @pl.when(wid < B)
            def _():
                b = wid
                pbase = b * N + full_per_b * CH
                pltpu.async_copy(px_hbm.at[pl.ds(pbase, tail)],
                                 pxv.at[0, pl.ds(0, tail)], psem)
                pltpu.async_copy(py_hbm.at[pl.ds(pbase, tail)],
                                 pyv.at[0, pl.ds(0, tail)], psem)
                pltpu.make_async_copy(px_hbm.at[pl.ds(0, tail)],
                                      pxv.at[0, pl.ds(0, tail)], psem).wait()
                pltpu.make_async_copy(py_hbm.at[pl.ds(0, tail)],
                                      pyv.at[0, pl.ds(0, tail)], psem).wait()
                weights_phase(0, tail)
                xb = x_hbm.at[b]
                for k in range(4):
                    pltpu.async_copy(
                        xb.at[idxv.at[0, k, pl.ds(0, tail)]],
                        rowsv.at[0, pl.ds(k * CH, tail)],
                        gsem,
                    )
                wait_gathers(tail)

                def tail_group(g, carry2):
                    gsl = pl.ds(g * L, L)
                    w00 = wv[0, 0, gsl]
                    w01 = wv[0, 1, gsl]
                    w10 = wv[0, 2, gsl]
                    w11 = wv[0, 3, gsl]
                    for j in range(L):
                        lanes = jnp.full((L,), j, jnp.int32)
                        b00 = w00.at[lanes].get(mode="promise_in_bounds")
                        b01 = w01.at[lanes].get(mode="promise_in_bounds")
                        b10 = w10.at[lanes].get(mode="promise_in_bounds")
                        b11 = w11.at[lanes].get(mode="promise_in_bounds")
                        p = g * L + j
                        for cc in range(C // L):
                            csl = pl.ds(cc * L, L)
                            acc = rowsv[0, p, csl] * b00
                            acc += rowsv[0, CH + p, csl] * b01
                            acc += rowsv[0, 2 * CH + p, csl] * b10
                            acc += rowsv[0, 3 * CH + p, csl] * b11
                            outv[0, p, csl] = acc
                    return carry2

                lax.fori_loop(0, tail // L, tail_group, 0)
                pltpu.sync_copy(
                    outv.at[0, pl.ds(0, tail)],
                    out_hbm.at[b, pl.ds(full_per_b * CH, tail)],
                )

    @jax.jit
    def run(x3, px, py):
        return interp(x3, px, py)

    return run


def kernel(x, pos, H, W):
    B, Hs, Ws, C = x.shape
    N = pos.shape[1]
    xp = jnp.concatenate(
        [jnp.pad(x[b : b + 1], ((0, 0), (0, 0), (0, 0), (0, CP - C)))
         for b in range(B)],
        axis=0,
    )
    x3 = xp.reshape(B, Hs * Ws, CP)
    px = pos[..., 0].reshape(-1)
    py = pos[..., 1].reshape(-1)
    return _build(B, Hs, Ws, C, N)(x3, px, py)


# speedup vs baseline: 1.2274x; 1.2274x over previous
import functools

import jax
import jax.numpy as jnp
from jax import lax
from jax.experimental import pallas as pl
from jax.experimental.pallas import tpu as pltpu
from jax.experimental.pallas import tpu_sc as plsc

NC = 2
NS = 16
NW = NC * NS
L = 16
CH = 96
CP = 128


@functools.cache
def _build(B, H, W, C, N):
    HW = H * W
    full_per_b = N // CH
    tail = N - full_per_b * CH
    n_main = B * full_per_b
    assert n_main % NW == 0, (n_main, NW)
    per_w = n_main // NW
    assert tail % 8 == 0 and tail < CH
    sx = float(W) / float(W - 1)
    sy = float(H) / float(H - 1)

    mesh = plsc.VectorSubcoreMesh(
        core_axis_name="c", subcore_axis_name="s", num_cores=NC, num_subcores=NS
    )

    @functools.partial(
        pl.kernel,
        out_type=jax.ShapeDtypeStruct((B, N, C), jnp.float32),
        mesh=mesh,
        scratch_types=[
            pltpu.VMEM((2, CH), jnp.float32),
            pltpu.VMEM((2, CH), jnp.float32),
            pltpu.VMEM((2, 4, CH), jnp.int32),
            pltpu.VMEM((2, 4, CH), jnp.float32),
            pltpu.VMEM((2, 4 * CH, CP), jnp.float32),
            pltpu.VMEM((2, CH, C), jnp.float32),
            pltpu.SemaphoreType.DMA,
            pltpu.SemaphoreType.DMA,
            pltpu.SemaphoreType.DMA,
        ],
        compiler_params=pltpu.CompilerParams(use_tc_tiling_on_sc=False),
    )
    def interp(x_hbm, px_hbm, py_hbm, out_hbm, pxv, pyv, idxv, wv, rowsv, outv,
               psem, gsem, osem):
        wid = lax.axis_index("s") * NC + lax.axis_index("c")
        c0 = wid * per_w

        def chunk_coords(c):
            b = jnp.int32(0)
            for bb in range(1, B):
                b = b + jnp.where(c >= bb * full_per_b, 1, 0)
            lc = c - b * full_per_b
            return b, lc

        def fire_pos(t):
            c = c0 + t
            pbase = chunk_coords(c)[0] * (N - full_per_b * CH) + c * CH
            buf = t & 1
            pltpu.async_copy(px_hbm.at[pl.ds(pbase, CH)], pxv.at[buf], psem)
            pltpu.async_copy(py_hbm.at[pl.ds(pbase, CH)], pyv.at[buf], psem)

        def wait_pos():
            pltpu.make_async_copy(px_hbm.at[pl.ds(0, CH)], pxv.at[0], psem).wait()
            pltpu.make_async_copy(py_hbm.at[pl.ds(0, CH)], pyv.at[0], psem).wait()

        def weights_phase(t, size):
            buf = t & 1
            for g in range(size // L):
                sl = pl.ds(g * L, L)
                px = pxv[buf, sl]
                py = pyv[buf, sl]
                ix = px * sx - 0.5
                iy = py * sy - 0.5
                fx0 = (ix + 1.0).astype(jnp.int32).astype(jnp.float32) - 1.0
                fy0 = (iy + 1.0).astype(jnp.int32).astype(jnp.float32) - 1.0
                wx1 = ix - fx0
                wx0 = 1.0 - wx1
                wy1 = iy - fy0
                wy0 = 1.0 - wy1
                fx1 = fx0 + 1.0
                fy1 = fy0 + 1.0
                mx0 = (fx0 >= 0.0) & (fx0 <= W - 1.0)
                mx1 = (fx1 >= 0.0) & (fx1 <= W - 1.0)
                my0 = (fy0 >= 0.0) & (fy0 <= H - 1.0)
                my1 = (fy1 >= 0.0) & (fy1 <= H - 1.0)
                cx0 = jnp.clip(fx0.astype(jnp.int32), 0, W - 1)
                cx1 = jnp.clip(fx1.astype(jnp.int32), 0, W - 1)
                cy0 = jnp.clip(fy0.astype(jnp.int32), 0, H - 1)
                cy1 = jnp.clip(fy1.astype(jnp.int32), 0, H - 1)
                r0 = cy0 * W
                r1 = cy1 * W
                idxv[buf, 0, sl] = r0 + cx0
                idxv[buf, 1, sl] = r0 + cx1
                idxv[buf, 2, sl] = r1 + cx0
                idxv[buf, 3, sl] = r1 + cx1
                zero = jnp.zeros((L,), jnp.float32)
                wv[buf, 0, sl] = jnp.where(mx0 & my0, wx0 * wy0, zero)
                wv[buf, 1, sl] = jnp.where(mx1 & my0, wx1 * wy0, zero)
                wv[buf, 2, sl] = jnp.where(mx0 & my1, wx0 * wy1, zero)
                wv[buf, 3, sl] = jnp.where(mx1 & my1, wx1 * wy1, zero)

        def fire_gathers(t, size):
            buf = t & 1
            b = chunk_coords(c0 + t)[0]
            xb = x_hbm.at[b]
            for k in range(4):
                pltpu.async_copy(
                    xb.at[idxv.at[buf, k, pl.ds(0, size)]],
                    rowsv.at[buf, pl.ds(k * CH, size)],
                    gsem,
                )

        def wait_gathers(size):
            for _ in range(4):
                pltpu.make_async_copy(
                    x_hbm.at[0, pl.ds(0, size)],
                    rowsv.at[0, pl.ds(0, size)],
                    gsem,
                ).wait()

        def compute_phase(t, size):
            buf = t & 1
            b, lc = chunk_coords(c0 + t)

            def group_body(g, carry2):
                gsl = pl.ds(g * L, L)
                w00 = wv[buf, 0, gsl]
                w01 = wv[buf, 1, gsl]
                w10 = wv[buf, 2, gsl]
                w11 = wv[buf, 3, gsl]
                for j in range(L):
                    lanes = jnp.full((L,), j, jnp.int32)
                    b00 = w00.at[lanes].get(mode="promise_in_bounds")
                    b01 = w01.at[lanes].get(mode="promise_in_bounds")
                    b10 = w10.at[lanes].get(mode="promise_in_bounds")
                    b11 = w11.at[lanes].get(mode="promise_in_bounds")
                    p = g * L + j
                    for cc in range(C // L):
                        csl = pl.ds(cc * L, L)
                        acc = rowsv[buf, p, csl] * b00
                        acc += rowsv[buf, CH + p, csl] * b01
                        acc += rowsv[buf, 2 * CH + p, csl] * b10
                        acc += rowsv[buf, 3 * CH + p, csl] * b11
                        outv[buf, p, csl] = acc
                return carry2

            lax.fori_loop(0, size // L, group_body, 0)
            pltpu.async_copy(
                outv.at[buf, pl.ds(0, size)],
                out_hbm.at[b, pl.ds(lc * CH, size)],
                osem,
            )

        def wait_out():
            pltpu.make_async_copy(
                out_hbm.at[0, pl.ds(0, CH)], outv.at[0], osem
            ).wait()

        fire_pos(0)

        def pipe_body(t, carry):
            @pl.when(t < per_w)
            def _():
                wait_pos()
                weights_phase(t, CH)

                @pl.when(t + 1 < per_w)
                def _():
                    fire_pos(t + 1)

            @pl.when(t >= 1)
            def _():
                wait_gathers(CH)

            @pl.when(t < per_w)
            def _():
                fire_gathers(t, CH)

            @pl.when(t >= 2)
            def _():
                wait_out()

            @pl.when(t >= 1)
            def _():
                compute_phase(t - 1, CH)

            return carry

        lax.fori_loop(0, per_w + 1, pipe_body, 0)
        wait_out()

        if tail:
            @pl.when(wid < B)
            def _():
                b = wid
                pbase = b * N + full_per_b * CH
                pltpu.async_copy(px_hbm.at[pl.ds(pbase, tail)],
                                 pxv.at[0, pl.ds(0, tail)], psem)
                pltpu.async_copy(py_hbm.at[pl.ds(pbase, tail)],
                                 pyv.at[0, pl.ds(0, tail)], psem)
                pltpu.make_async_copy(px_hbm.at[pl.ds(0, tail)],
                                      pxv.at[0, pl.ds(0, tail)], psem).wait()
                pltpu.make_async_copy(py_hbm.at[pl.ds(0, tail)],
                                      pyv.at[0, pl.ds(0, tail)], psem).wait()
                weights_phase(0, tail)
                xb = x_hbm.at[b]
                for k in range(4):
                    pltpu.async_copy(
                        xb.at[idxv.at[0, k, pl.ds(0, tail)]],
                        rowsv.at[0, pl.ds(k * CH, tail)],
                        gsem,
                    )
                wait_gathers(tail)

                def tail_group(g, carry2):
                    gsl = pl.ds(g * L, L)
                    w00 = wv[0, 0, gsl]
                    w01 = wv[0, 1, gsl]
                    w10 = wv[0, 2, gsl]
                    w11 = wv[0, 3, gsl]
                    for j in range(L):
                        lanes = jnp.full((L,), j, jnp.int32)
                        b00 = w00.at[lanes].get(mode="promise_in_bounds")
                        b01 = w01.at[lanes].get(mode="promise_in_bounds")
                        b10 = w10.at[lanes].get(mode="promise_in_bounds")
                        b11 = w11.at[lanes].get(mode="promise_in_bounds")
                        p = g * L + j
                        for cc in range(C // L):
                            csl = pl.ds(cc * L, L)
                            acc = rowsv[0, p, csl] * b00
                            acc += rowsv[0, CH + p, csl] * b01
                            acc += rowsv[0, 2 * CH + p, csl] * b10
                            acc += rowsv[0, 3 * CH + p, csl] * b11
                            outv[0, p, csl] = acc
                    return carry2

                lax.fori_loop(0, tail // L, tail_group, 0)
                pltpu.sync_copy(
                    outv.at[0, pl.ds(0, tail)],
                    out_hbm.at[b, pl.ds(full_per_b * CH, tail)],
                )

    @jax.jit
    def run(x3, px, py):
        return interp(x3, px, py)

    return run


def kernel(x, pos, H, W):
    B, Hs, Ws, C = x.shape
    N = pos.shape[1]
    xp = jnp.pad(x, ((0, 0), (0, 0), (0, 0), (0, CP - C)))
    x3 = xp.reshape(B, Hs * Ws, CP)
    px = pos[..., 0].reshape(-1)
    py = pos[..., 1].reshape(-1)
    return _build(B, Hs, Ws, C, N)(x3, px, py)
